# baseline (speedup 1.0000x reference)
import functools

import jax
import jax.numpy as jnp
from jax import lax
from jax.experimental import pallas as pl
from jax.experimental.pallas import tpu as pltpu
from jax.experimental.pallas import tpu_sc as plsc

N = 10000
E = 160000
NP = 10240
EP = 163840
EH = EP // 2
NC, NS = 2, 16
NW = NC * NS
CHUNK = 128
NCH = EH // (NW * CHUNK)
EPW = NCH * CHUNK
ROWS_PT = NP // NS

BLK_E = 4096
BLK_N = 2048

NB = 5
NSB = 2

_mesh = plsc.VectorSubcoreMesh(core_axis_name="c", subcore_axis_name="s",
                               num_cores=NC, num_subcores=NS)


@functools.partial(
    pl.kernel,
    out_type=jax.ShapeDtypeStruct((EH, 128), jnp.float32),
    mesh=_mesh,
    scratch_types=[
        pltpu.VMEM((NCH, CHUNK), jnp.int32),
        pltpu.VMEM((NCH, CHUNK), jnp.int32),
        pltpu.VMEM((NB, CHUNK, 128), jnp.float32),
    ] + [pltpu.SemaphoreType.DMA] * (3 * NB),
)
def _sc_gather(s_hbm, b_hbm, src_hbm, dst_hbm, z_out, src_v, dst_v, buf,
               *sems):
    sg = sems[0:NB]
    sb = sems[NB:2 * NB]
    so = sems[2 * NB:3 * NB]
    c = lax.axis_index("c")
    s = lax.axis_index("s")
    w = c * NS + s
    pltpu.sync_copy(src_hbm.at[w], src_v)
    pltpu.sync_copy(dst_hbm.at[w], dst_v)

    @pl.loop(0, NCH + 2 + NB, step=NB)
    def _steps(g):
        for b in range(NB):
            t = g + b
            bA = b
            bB = (b - 1) % NB
            bC = (b - 2) % NB

            @pl.when(t < NCH)
            def _():
                @pl.when(t >= NB)
                def _():
                    pltpu.make_async_copy(
                        buf.at[bA], z_out.at[pl.ds(0, CHUNK)], so[bA]).wait()
                pltpu.async_copy(s_hbm.at[src_v.at[t]], buf.at[bA], sg[bA])

            @pl.when(jnp.logical_and(t >= 1, t <= NCH))
            def _():
                pltpu.make_async_copy(
                    s_hbm.at[src_v.at[0]], buf.at[bB], sg[bB]).wait()
                pltpu.async_copy(b_hbm.at[dst_v.at[t - 1]], buf.at[bB],
                                 sb[bB], add=True)

            @pl.when(jnp.logical_and(t >= 2, t <= NCH + 1))
            def _():
                pltpu.make_async_copy(
                    b_hbm.at[dst_v.at[0]], buf.at[bC], sb[bC]).wait()
                base = w * EPW + (t - 2) * CHUNK
                pltpu.async_copy(buf.at[bC], z_out.at[pl.ds(base, CHUNK)],
                                 so[bC])

    for b in range(NB):
        pltpu.make_async_copy(buf.at[b], z_out.at[pl.ds(0, CHUNK)],
                              so[b]).wait()


def _scatter_pipe(m_hbm, dst_v, m_v, acc, sl, ss, w, d0):
    @pl.loop(0, NCH + NSB, step=NSB)
    def _steps(g):
        for b in range(NSB):
            t = g + b
            bB = (b - 1) % NSB

            @pl.when(t < NCH)
            def _():
                @pl.when(t >= NSB)
                def _():
                    pltpu.make_async_copy(m_v.at[b], acc.at[dst_v.at[0]],
                                          ss[b]).wait()
                base = w * EPW + t * CHUNK
                pltpu.async_copy(m_hbm.at[pl.ds(base, CHUNK)], m_v.at[b],
                                 sl[b])

            @pl.when(jnp.logical_and(t >= 1, t <= NCH))
            def _():
                pltpu.make_async_copy(m_hbm.at[pl.ds(0, CHUNK)],
                                      m_v.at[bB], sl[bB]).wait()
                pltpu.async_copy(m_v.at[bB], acc.at[dst_v.at[d0 + t - 1]],
                                 ss[bB], add=True)

    for b in range(NSB):
        pltpu.make_async_copy(m_v.at[b], acc.at[dst_v.at[0]], ss[b]).wait()


@functools.partial(
    pl.kernel,
    out_type=jax.ShapeDtypeStruct((NC, NP, 128), jnp.float32),
    mesh=_mesh,
    scratch_types=[
        pltpu.VMEM((2 * NCH, CHUNK), jnp.int32),
        pltpu.VMEM((NSB, CHUNK, 128), jnp.float32),
        pltpu.VMEM_SHARED((NP, 128), jnp.float32),
    ] + [pltpu.SemaphoreType.DMA] * (2 * NSB),
)
def _sc_scatter(m1_hbm, m2_hbm, dst1_hbm, dst2_hbm, z_hbm, p_out,
                dst_v, m_v, acc, *sems):
    sl = sems[0:NSB]
    ss = sems[NSB:2 * NSB]
    c = lax.axis_index("c")
    s = lax.axis_index("s")
    w = c * NS + s
    pltpu.sync_copy(z_hbm.at[pl.ds(s * ROWS_PT, ROWS_PT)],
                    acc.at[pl.ds(s * ROWS_PT, ROWS_PT)])
    pltpu.sync_copy(dst1_hbm.at[w], dst_v.at[pl.ds(0, NCH)])
    pltpu.sync_copy(dst2_hbm.at[w], dst_v.at[pl.ds(NCH, NCH)])
    plsc.subcore_barrier()
    _scatter_pipe(m1_hbm, dst_v, m_v, acc, sl, ss, w, 0)
    _scatter_pipe(m2_hbm, dst_v, m_v, acc, sl, ss, w, NCH)
    plsc.subcore_barrier()
    pltpu.sync_copy(acc.at[pl.ds(s * ROWS_PT, ROWS_PT)],
                    p_out.at[c, pl.ds(s * ROWS_PT, ROWS_PT)])


def _relu(v):
    return jnp.maximum(v, 0.0)


def _full(shape):
    return pl.BlockSpec(shape, lambda i: (0,) * len(shape))


def _node_enc_body(x8, f1w, f1b, f2w, f2b, wtx, wts, tabb, x0_o, s_o, b_o):
    h = _relu(x8[...] @ f1w[...] + f1b[...])
    x0 = _relu(h @ f2w[...] + f2b[...])
    x0_o[...] = x0
    t = x0 @ wtx[...] + x0 @ wts[...] + tabb[...]
    s_o[...] = t[:, :128]
    b_o[...] = t[:, 128:]


def _edge_enc_body(e8, w1, b1, w2, b2, we2, em1b, ea_o, q_o):
    h = _relu(e8[...] @ w1[...] + b1[...])
    ea0 = _relu(h @ w2[...] + b2[...])
    ea_o[...] = ea0
    q_o[...] = ea0 @ we2[...] + em1b[...]


def _node_round_body(x, sx, p, w21x, w21s, w21g, b21, w22, b22, wtx, wts,
                     tabb, x_o, s_o, b_o):
    agg = (p[0] + p[1])[:, :64]
    h = _relu(x[...] @ w21x[...] + sx[...] @ w21s[...] + agg @ w21g[...]
              + b21[...])
    x2 = _relu(h @ w22[...] + b22[...])
    x_o[...] = x2
    t = x2 @ wtx[...] + sx[...] @ wts[...] + tabb[...]
    s_o[...] = t[:, :128]
    b_o[...] = t[:, 128:]


def _dec_body(x, f3w, f3b, f4w, f4b, f5w, f5b, o):
    h = _relu(x[...] @ f3w[...] + f3b[...])
    h = _relu(h @ f4w[...] + f4b[...])
    o[...] = jnp.sum(h * f5w[...], axis=1, keepdims=True) + f5b[...]


def _node_enc(x8, f1w, f1b, f2w, f2b, wtx, wts, tabb):
    return pl.pallas_call(
        _node_enc_body,
        grid=(NP // BLK_N,),
        in_specs=[pl.BlockSpec((BLK_N, 8), lambda i: (i, 0)),
                  _full((8, 64)), _full((1, 64)), _full((64, 64)),
                  _full((1, 64)), _full((64, 256)), _full((64, 256)),
                  _full((1, 256))],
        out_specs=[pl.BlockSpec((BLK_N, 64), lambda i: (i, 0)),
                   pl.BlockSpec((BLK_N, 128), lambda i: (i, 0)),
                   pl.BlockSpec((BLK_N, 128), lambda i: (i, 0))],
        out_shape=[jax.ShapeDtypeStruct((NP, 64), jnp.float32),
                   jax.ShapeDtypeStruct((NP, 128), jnp.float32),
                   jax.ShapeDtypeStruct((NP, 128), jnp.float32)],
    )(x8, f1w, f1b, f2w, f2b, wtx, wts, tabb)


def _edge_enc(e8, w1, b1, w2, b2, we2, em1b):
    return pl.pallas_call(
        _edge_enc_body,
        grid=(EH // BLK_E,),
        in_specs=[pl.BlockSpec((BLK_E, 8), lambda i: (i, 0)),
                  _full((8, 64)), _full((1, 64)), _full((64, 64)),
                  _full((1, 64)), _full((64, 64)), _full((1, 64))],
        out_specs=[pl.BlockSpec((BLK_E, 64), lambda i: (i, 0)),
                   pl.BlockSpec((BLK_E, 64), lambda i: (i, 0))],
        out_shape=[jax.ShapeDtypeStruct((EH, 64), jnp.float32),
                   jax.ShapeDtypeStruct((EH, 64), jnp.float32)],
    )(e8, w1, b1, w2, b2, we2, em1b)


def _make_edge_round(row0):
    def body(gs, ea, q, we1, em2w, em2b, wne, n12w, n12b, ea_o, m_o):
        g = gs[...]
        h = _relu(g[:, :64] + ea[...] @ we1[...] + q[...])
        ea2 = _relu(h @ em2w[...] + em2b[...])
        ea_o[...] = ea2
        h2 = _relu(g[:, 64:] + ea2 @ wne[...])
        m = _relu(h2 @ n12w[...] + n12b[...])
        i = pl.program_id(0)
        rows = row0 + i * BLK_E + lax.broadcasted_iota(jnp.int32,
                                                       (BLK_E, 1), 0)
        m = jnp.where(rows < E, m, 0.0)
        m_o[...] = jnp.concatenate([m, jnp.zeros_like(m)], axis=1)

    def call(gs, ea, q, we1, em2w, em2b, wne, n12w, n12b):
        return pl.pallas_call(
            body,
            grid=(EH // BLK_E,),
            in_specs=[pl.BlockSpec((BLK_E, 128), lambda i: (i, 0)),
                      pl.BlockSpec((BLK_E, 64), lambda i: (i, 0)),
                      pl.BlockSpec((BLK_E, 64), lambda i: (i, 0)),
                      _full((64, 64)), _full((64, 64)), _full((1, 64)),
                      _full((64, 64)), _full((64, 64)), _full((1, 64))],
            out_specs=[pl.BlockSpec((BLK_E, 64), lambda i: (i, 0)),
                       pl.BlockSpec((BLK_E, 128), lambda i: (i, 0))],
            out_shape=[jax.ShapeDtypeStruct((EH, 64), jnp.float32),
                       jax.ShapeDtypeStruct((EH, 128), jnp.float32)],
        )(gs, ea, q, we1, em2w, em2b, wne, n12w, n12b)

    return call


_edge_round_1 = _make_edge_round(0)
_edge_round_2 = _make_edge_round(EH)


def _node_round(x, sx, p, w21x, w21s, w21g, b21, w22, b22, wtx, wts, tabb):
    return pl.pallas_call(
        _node_round_body,
        grid=(NP // BLK_N,),
        in_specs=[pl.BlockSpec((BLK_N, 64), lambda i: (i, 0)),
                  pl.BlockSpec((BLK_N, 64), lambda i: (i, 0)),
                  pl.BlockSpec((NC, BLK_N, 128), lambda i: (0, i, 0)),
                  _full((64, 64)), _full((64, 64)), _full((64, 64)),
                  _full((1, 64)), _full((64, 64)), _full((1, 64)),
                  _full((64, 256)), _full((64, 256)), _full((1, 256))],
        out_specs=[pl.BlockSpec((BLK_N, 64), lambda i: (i, 0)),
                   pl.BlockSpec((BLK_N, 128), lambda i: (i, 0)),
                   pl.BlockSpec((BLK_N, 128), lambda i: (i, 0))],
        out_shape=[jax.ShapeDtypeStruct((NP, 64), jnp.float32),
                   jax.ShapeDtypeStruct((NP, 128), jnp.float32),
                   jax.ShapeDtypeStruct((NP, 128), jnp.float32)],
    )(x, sx, p, w21x, w21s, w21g, b21, w22, b22, wtx, wts, tabb)


def _decoder(x, f3w, f3b, f4w, f4b, f5w, f5b):
    return pl.pallas_call(
        _dec_body,
        grid=(NP // BLK_N,),
        in_specs=[pl.BlockSpec((BLK_N, 64), lambda i: (i, 0)),
                  _full((64, 256)), _full((1, 256)), _full((256, 256)),
                  _full((1, 256)), _full((1, 256)), _full((1, 1))],
        out_specs=pl.BlockSpec((BLK_N, 1), lambda i: (i, 0)),
        out_shape=jax.ShapeDtypeStruct((NP, 1), jnp.float32),
    )(x, f3w, f3b, f4w, f4b, f5w, f5b)


def kernel(x, edge_index, edge_attr, fc1_w, fc1_b, fc2_w, fc2_b, efc1_w,
           efc1_b, efc2_w, efc2_b, em1_w, em1_b, em2_w, em2_b, nm11_w,
           nm11_b, nm12_w, nm12_b, nm21_w, nm21_b, nm22_w, nm22_b, fc3_w,
           fc3_b, fc4_w, fc4_b, fc5_w, fc5_b):
    f32 = jnp.float32
    row = lambda b: b.reshape(1, -1).astype(f32)

    x8 = jnp.pad(x, ((0, NP - N), (0, 5)))
    e8 = jnp.pad(edge_attr, ((0, EP - E), (0, 6)))
    f1w = jnp.pad(fc1_w.T, ((0, 5), (0, 0)))
    ef1w = jnp.pad(efc1_w.T, ((0, 6), (0, 0)))

    src_p = jnp.pad(edge_index[0], (0, EP - E)).astype(jnp.int32)
    dst_p = jnp.pad(edge_index[1], (0, EP - E)).astype(jnp.int32)
    src1 = src_p[:EH].reshape(NW, NCH, CHUNK)
    src2 = src_p[EH:].reshape(NW, NCH, CHUNK)
    dst1 = dst_p[:EH].reshape(NW, NCH, CHUNK)
    dst2 = dst_p[EH:].reshape(NW, NCH, CHUNK)

    em1T = em1_w.T
    w_src, w_dst = em1T[:128], em1T[128:256]
    w_e1, w_e2 = em1T[256:320], em1T[320:384]
    nm11T = nm11_w.T
    w_ns, w_ne = nm11T[:128], nm11T[128:]
    w_tab = jnp.concatenate(
        [w_src, w_ns, w_dst, jnp.zeros((128, 64), f32)], axis=1)
    wtx, wts = w_tab[:64], w_tab[64:]
    tabb = jnp.concatenate(
        [jnp.zeros((1, 64), f32), row(nm11_b), jnp.zeros((1, 128), f32)],
        axis=1)
    nm21T = nm21_w.T
    w21x, w21s, w21g = nm21T[:64], nm21T[64:128], nm21T[128:]

    zeros_n = jnp.zeros((NP, 128), f32)

    x0, S, B = _node_enc(x8, f1w, row(fc1_b), fc2_w.T, row(fc2_b),
                         wtx, wts, tabb)
    ea1, q1 = _edge_enc(e8[:EH], ef1w, row(efc1_b), efc2_w.T, row(efc2_b),
                        w_e2, row(em1_b))
    ea2, q2 = _edge_enc(e8[EH:], ef1w, row(efc1_b), efc2_w.T, row(efc2_b),
                        w_e2, row(em1_b))

    sx = x0
    xcur = x0
    em2T, n12T = em2_w.T, nm12_w.T
    for _ in range(7):
        z1 = _sc_gather(S, B, src1, dst1)
        z2 = _sc_gather(S, B, src2, dst2)
        ea1, m1 = _edge_round_1(z1, ea1, q1, w_e1, em2T, row(em2_b),
                                w_ne, n12T, row(nm12_b))
        ea2, m2 = _edge_round_2(z2, ea2, q2, w_e1, em2T, row(em2_b),
                                w_ne, n12T, row(nm12_b))
        p = _sc_scatter(m1, m2, dst1, dst2, zeros_n)
        xcur, S, B = _node_round(xcur, sx, p, w21x, w21s, w21g,
                                 row(nm21_b), nm22_w.T, row(nm22_b),
                                 wtx, wts, tabb)

    out = _decoder(xcur, fc3_w.T, row(fc3_b), fc4_w.T, row(fc4_b),
                   row(fc5_w), fc5_b.reshape(1, 1))
    return out[:N]


# speedup vs baseline: 1.0248x; 1.0248x over previous
import functools

import jax
import jax.numpy as jnp
from jax import lax
from jax.experimental import pallas as pl
from jax.experimental.pallas import tpu as pltpu
from jax.experimental.pallas import tpu_sc as plsc

N = 10000
E = 160000
NP = 10240
EP = 163840
NC, NS = 2, 16
NW = NC * NS
EPW = EP // NW
CHUNK = 128
NCHUNK = EPW // CHUNK
ROWS_PT = NP // NS

BLK_E = 4096
BLK_N = 2048

_mesh = plsc.VectorSubcoreMesh(core_axis_name="c", subcore_axis_name="s",
                               num_cores=NC, num_subcores=NS)


NB = 5

@functools.partial(
    pl.kernel,
    out_type=jax.ShapeDtypeStruct((EP, 128), jnp.float32),
    mesh=_mesh,
    scratch_types=[
        pltpu.VMEM((NCHUNK, CHUNK), jnp.int32),
        pltpu.VMEM((NCHUNK, CHUNK), jnp.int32),
        pltpu.VMEM((NB, CHUNK, 128), jnp.float32),
    ] + [pltpu.SemaphoreType.DMA] * (3 * NB),
)
def _sc_gather(s_hbm, b_hbm, src_hbm, dst_hbm, z_out, src_v, dst_v, buf,
               *sems):
    sg = sems[0:NB]
    sb = sems[NB:2 * NB]
    so = sems[2 * NB:3 * NB]
    c = lax.axis_index("c")
    s = lax.axis_index("s")
    w = c * NS + s
    pltpu.sync_copy(src_hbm.at[w], src_v)
    pltpu.sync_copy(dst_hbm.at[w], dst_v)

    @pl.loop(0, NCHUNK + 2 + NB, step=NB)
    def _steps(g):
        for b in range(NB):
            t = g + b
            bA = b
            bB = (b - 1) % NB
            bC = (b - 2) % NB

            @pl.when(t < NCHUNK)
            def _():
                @pl.when(t >= NB)
                def _():
                    pltpu.make_async_copy(
                        buf.at[bA], z_out.at[pl.ds(0, CHUNK)], so[bA]).wait()
                pltpu.async_copy(s_hbm.at[src_v.at[t]], buf.at[bA], sg[bA])

            @pl.when(jnp.logical_and(t >= 1, t <= NCHUNK))
            def _():
                pltpu.make_async_copy(
                    s_hbm.at[src_v.at[0]], buf.at[bB], sg[bB]).wait()
                pltpu.async_copy(b_hbm.at[dst_v.at[t - 1]], buf.at[bB],
                                 sb[bB], add=True)

            @pl.when(jnp.logical_and(t >= 2, t <= NCHUNK + 1))
            def _():
                pltpu.make_async_copy(
                    b_hbm.at[dst_v.at[0]], buf.at[bC], sb[bC]).wait()
                base = w * EPW + (t - 2) * CHUNK
                pltpu.async_copy(buf.at[bC], z_out.at[pl.ds(base, CHUNK)],
                                 so[bC])

    for b in range(NB):
        pltpu.make_async_copy(buf.at[b], z_out.at[pl.ds(0, CHUNK)],
                              so[b]).wait()


NSB = 2


@functools.partial(
    pl.kernel,
    out_type=jax.ShapeDtypeStruct((NC, NP, 128), jnp.float32),
    mesh=_mesh,
    scratch_types=[
        pltpu.VMEM((NCHUNK, CHUNK), jnp.int32),
        pltpu.VMEM((NSB, CHUNK, 128), jnp.float32),
        pltpu.VMEM_SHARED((NP, 128), jnp.float32),
    ] + [pltpu.SemaphoreType.DMA] * (2 * NSB),
)
def _sc_scatter(m_hbm, dst_hbm, z_hbm, p_out, dst_v, m_v, acc, *sems):
    sl = sems[0:NSB]
    ss = sems[NSB:2 * NSB]
    c = lax.axis_index("c")
    s = lax.axis_index("s")
    w = c * NS + s
    pltpu.sync_copy(z_hbm.at[pl.ds(s * ROWS_PT, ROWS_PT)],
                    acc.at[pl.ds(s * ROWS_PT, ROWS_PT)])
    pltpu.sync_copy(dst_hbm.at[w], dst_v)
    plsc.subcore_barrier()

    @pl.loop(0, NCHUNK + NSB, step=NSB)
    def _steps(g):
        for b in range(NSB):
            t = g + b
            bB = (b - 1) % NSB

            @pl.when(t < NCHUNK)
            def _():
                @pl.when(t >= NSB)
                def _():
                    pltpu.make_async_copy(m_v.at[b], acc.at[dst_v.at[0]],
                                          ss[b]).wait()
                base = w * EPW + t * CHUNK
                pltpu.async_copy(m_hbm.at[pl.ds(base, CHUNK)], m_v.at[b],
                                 sl[b])

            @pl.when(jnp.logical_and(t >= 1, t <= NCHUNK))
            def _():
                pltpu.make_async_copy(m_hbm.at[pl.ds(0, CHUNK)],
                                      m_v.at[bB], sl[bB]).wait()
                pltpu.async_copy(m_v.at[bB], acc.at[dst_v.at[t - 1]],
                                 ss[bB], add=True)

    for b in range(NSB):
        pltpu.make_async_copy(m_v.at[b], acc.at[dst_v.at[0]], ss[b]).wait()
    plsc.subcore_barrier()
    pltpu.sync_copy(acc.at[pl.ds(s * ROWS_PT, ROWS_PT)],
                    p_out.at[c, pl.ds(s * ROWS_PT, ROWS_PT)])


def _relu(v):
    return jnp.maximum(v, 0.0)


def _full(shape):
    return pl.BlockSpec(shape, lambda i: (0,) * len(shape))


def _node_enc_body(x8, f1w, f1b, f2w, f2b, wtx, wts, tabb, x0_o, s_o, b_o):
    h = _relu(x8[...] @ f1w[...] + f1b[...])
    x0 = _relu(h @ f2w[...] + f2b[...])
    x0_o[...] = x0
    t = x0 @ wtx[...] + x0 @ wts[...] + tabb[...]
    s_o[...] = t[:, :128]
    b_o[...] = t[:, 128:]


def _edge_enc_body(e8, w1, b1, w2, b2, we2, em1b, ea_o, q_o):
    h = _relu(e8[...] @ w1[...] + b1[...])
    ea0 = _relu(h @ w2[...] + b2[...])
    ea_o[...] = ea0
    q_o[...] = ea0 @ we2[...] + em1b[...]


def _edge_round_body(gs, ea, q, we1, em2w, em2b, wne, n12w, n12b,
                     ea_o, m_o):
    g = gs[...]
    h = _relu(g[:, :64] + ea[...] @ we1[...] + q[...])
    ea2 = _relu(h @ em2w[...] + em2b[...])
    ea_o[...] = ea2
    h2 = _relu(g[:, 64:] + ea2 @ wne[...])
    m = _relu(h2 @ n12w[...] + n12b[...])
    i = pl.program_id(0)
    rows = i * BLK_E + lax.broadcasted_iota(jnp.int32, (BLK_E, 1), 0)
    m = jnp.where(rows < E, m, 0.0)
    m_o[...] = jnp.concatenate([m, jnp.zeros_like(m)], axis=1)


def _node_round_body(x, sx, p, w21x, w21s, w21g, b21, w22, b22, wtx, wts,
                     tabb, x_o, s_o, b_o):
    agg = (p[0] + p[1])[:, :64]
    h = _relu(x[...] @ w21x[...] + sx[...] @ w21s[...] + agg @ w21g[...]
              + b21[...])
    x2 = _relu(h @ w22[...] + b22[...])
    x_o[...] = x2
    t = x2 @ wtx[...] + sx[...] @ wts[...] + tabb[...]
    s_o[...] = t[:, :128]
    b_o[...] = t[:, 128:]


def _dec_body(x, f3w, f3b, f4w, f4b, f5w, f5b, o):
    h = _relu(x[...] @ f3w[...] + f3b[...])
    h = _relu(h @ f4w[...] + f4b[...])
    o[...] = jnp.sum(h * f5w[...], axis=1, keepdims=True) + f5b[...]


def _node_enc(x8, f1w, f1b, f2w, f2b, wtx, wts, tabb):
    return pl.pallas_call(
        _node_enc_body,
        grid=(NP // BLK_N,),
        in_specs=[pl.BlockSpec((BLK_N, 8), lambda i: (i, 0)),
                  _full((8, 64)), _full((1, 64)), _full((64, 64)),
                  _full((1, 64)), _full((64, 256)), _full((64, 256)),
                  _full((1, 256))],
        out_specs=[pl.BlockSpec((BLK_N, 64), lambda i: (i, 0)),
                   pl.BlockSpec((BLK_N, 128), lambda i: (i, 0)),
                   pl.BlockSpec((BLK_N, 128), lambda i: (i, 0))],
        out_shape=[jax.ShapeDtypeStruct((NP, 64), jnp.float32),
                   jax.ShapeDtypeStruct((NP, 128), jnp.float32),
                   jax.ShapeDtypeStruct((NP, 128), jnp.float32)],
    )(x8, f1w, f1b, f2w, f2b, wtx, wts, tabb)


def _edge_enc(e8, w1, b1, w2, b2, we2, em1b):
    return pl.pallas_call(
        _edge_enc_body,
        grid=(EP // BLK_E,),
        in_specs=[pl.BlockSpec((BLK_E, 8), lambda i: (i, 0)),
                  _full((8, 64)), _full((1, 64)), _full((64, 64)),
                  _full((1, 64)), _full((64, 64)), _full((1, 64))],
        out_specs=[pl.BlockSpec((BLK_E, 64), lambda i: (i, 0)),
                   pl.BlockSpec((BLK_E, 64), lambda i: (i, 0))],
        out_shape=[jax.ShapeDtypeStruct((EP, 64), jnp.float32),
                   jax.ShapeDtypeStruct((EP, 64), jnp.float32)],
    )(e8, w1, b1, w2, b2, we2, em1b)


def _edge_round(gs2, ea, q, we1, em2w, em2b, wne, n12w, n12b):
    return pl.pallas_call(
        _edge_round_body,
        grid=(EP // BLK_E,),
        in_specs=[pl.BlockSpec((BLK_E, 128), lambda i: (i, 0)),
                  pl.BlockSpec((BLK_E, 64), lambda i: (i, 0)),
                  pl.BlockSpec((BLK_E, 64), lambda i: (i, 0)),
                  _full((64, 64)), _full((64, 64)), _full((1, 64)),
                  _full((64, 64)), _full((64, 64)), _full((1, 64))],
        out_specs=[pl.BlockSpec((BLK_E, 64), lambda i: (i, 0)),
                   pl.BlockSpec((BLK_E, 128), lambda i: (i, 0))],
        out_shape=[jax.ShapeDtypeStruct((EP, 64), jnp.float32),
                   jax.ShapeDtypeStruct((EP, 128), jnp.float32)],
    )(gs2, ea, q, we1, em2w, em2b, wne, n12w, n12b)


def _node_round(x, sx, p, w21x, w21s, w21g, b21, w22, b22, wtx, wts, tabb):
    return pl.pallas_call(
        _node_round_body,
        grid=(NP // BLK_N,),
        in_specs=[pl.BlockSpec((BLK_N, 64), lambda i: (i, 0)),
                  pl.BlockSpec((BLK_N, 64), lambda i: (i, 0)),
                  pl.BlockSpec((NC, BLK_N, 128), lambda i: (0, i, 0)),
                  _full((64, 64)), _full((64, 64)), _full((64, 64)),
                  _full((1, 64)), _full((64, 64)), _full((1, 64)),
                  _full((64, 256)), _full((64, 256)), _full((1, 256))],
        out_specs=[pl.BlockSpec((BLK_N, 64), lambda i: (i, 0)),
                   pl.BlockSpec((BLK_N, 128), lambda i: (i, 0)),
                   pl.BlockSpec((BLK_N, 128), lambda i: (i, 0))],
        out_shape=[jax.ShapeDtypeStruct((NP, 64), jnp.float32),
                   jax.ShapeDtypeStruct((NP, 128), jnp.float32),
                   jax.ShapeDtypeStruct((NP, 128), jnp.float32)],
    )(x, sx, p, w21x, w21s, w21g, b21, w22, b22, wtx, wts, tabb)


def _decoder(x, f3w, f3b, f4w, f4b, f5w, f5b):
    return pl.pallas_call(
        _dec_body,
        grid=(NP // BLK_N,),
        in_specs=[pl.BlockSpec((BLK_N, 64), lambda i: (i, 0)),
                  _full((64, 256)), _full((1, 256)), _full((256, 256)),
                  _full((1, 256)), _full((1, 256)), _full((1, 1))],
        out_specs=pl.BlockSpec((BLK_N, 1), lambda i: (i, 0)),
        out_shape=jax.ShapeDtypeStruct((NP, 1), jnp.float32),
    )(x, f3w, f3b, f4w, f4b, f5w, f5b)


def kernel(x, edge_index, edge_attr, fc1_w, fc1_b, fc2_w, fc2_b, efc1_w,
           efc1_b, efc2_w, efc2_b, em1_w, em1_b, em2_w, em2_b, nm11_w,
           nm11_b, nm12_w, nm12_b, nm21_w, nm21_b, nm22_w, nm22_b, fc3_w,
           fc3_b, fc4_w, fc4_b, fc5_w, fc5_b):
    f32 = jnp.float32
    row = lambda b: b.reshape(1, -1).astype(f32)

    x8 = jnp.pad(x, ((0, NP - N), (0, 5)))
    e8 = jnp.pad(edge_attr, ((0, EP - E), (0, 6)))
    f1w = jnp.pad(fc1_w.T, ((0, 5), (0, 0)))
    ef1w = jnp.pad(efc1_w.T, ((0, 6), (0, 0)))

    src = jnp.pad(edge_index[0], (0, EP - E)).reshape(
        NW, NCHUNK, CHUNK).astype(jnp.int32)
    dst = jnp.pad(edge_index[1], (0, EP - E)).reshape(
        NW, NCHUNK, CHUNK).astype(jnp.int32)

    em1T = em1_w.T
    w_src, w_dst = em1T[:128], em1T[128:256]
    w_e1, w_e2 = em1T[256:320], em1T[320:384]
    nm11T = nm11_w.T
    w_ns, w_ne = nm11T[:128], nm11T[128:]
    w_tab = jnp.concatenate(
        [w_src, w_ns, w_dst, jnp.zeros((128, 64), f32)], axis=1)
    wtx, wts = w_tab[:64], w_tab[64:]
    tabb = jnp.concatenate(
        [jnp.zeros((1, 64), f32), row(nm11_b), jnp.zeros((1, 128), f32)],
        axis=1)
    nm21T = nm21_w.T
    w21x, w21s, w21g = nm21T[:64], nm21T[64:128], nm21T[128:]

    zeros_n = jnp.zeros((NP, 128), f32)

    x0, S, B = _node_enc(x8, f1w, row(fc1_b), fc2_w.T, row(fc2_b),
                         wtx, wts, tabb)
    ea, q = _edge_enc(e8, ef1w, row(efc1_b), efc2_w.T, row(efc2_b),
                      w_e2, row(em1_b))

    sx = x0
    xcur = x0
    for _ in range(7):
        gs = _sc_gather(S, B, src, dst)
        ea, m = _edge_round(gs, ea, q, w_e1,
                            em2_w.T, row(em2_b), w_ne, nm12_w.T,
                            row(nm12_b))
        p = _sc_scatter(m, dst, zeros_n)
        xcur, S, B = _node_round(xcur, sx, p, w21x, w21s, w21g,
                                 row(nm21_b), nm22_w.T, row(nm22_b),
                                 wtx, wts, tabb)

    out = _decoder(xcur, fc3_w.T, row(fc3_b), fc4_w.T, row(fc4_b),
                   row(fc5_w), fc5_b.reshape(1, 1))
    return out[:N]
